# Initial kernel scaffold; baseline (speedup 1.0000x reference)
import jax
import jax.numpy as jnp
from jax.experimental import pallas as pl


def kernel(logits):
    raise NotImplementedError("write your pallas kernel here")


# speedup vs baseline: 161.1275x; 161.1275x over previous
import jax
import jax.numpy as jnp
from jax.experimental import pallas as pl

_TOP_P = 0.9
_TOP_K = 50
_ROWS_PER_BLOCK = 8


def _sampler_block(x_ref, conf_ref, idx_ref, conf2_ref):
    x = x_ref[...]
    l_max = jnp.max(x, axis=1, keepdims=True)
    e = jnp.exp(x - l_max)
    z = jnp.sum(e, axis=1, keepdims=True)
    pz = _TOP_P * z

    kb = jax.lax.bitcast_convert_type(x, jnp.uint32)
    key = jnp.where(
        kb >= jnp.uint32(0x80000000),
        jnp.bitwise_not(kb),
        jnp.bitwise_or(kb, jnp.uint32(0x80000000)),
    )

    def _bs(_, lohi):
        lo, hi = lohi
        mid = lo + ((hi - lo) >> jnp.uint32(1))
        cnt = jnp.sum((key >= mid).astype(jnp.int32), axis=1, keepdims=True)
        ge = cnt >= _TOP_K
        return jnp.where(ge, mid, lo), jnp.where(ge, hi, mid)

    r = x.shape[0]
    lo0 = jnp.zeros((r, 1), jnp.uint32)
    hi0 = jnp.full((r, 1), 0xFFFFFFFF, jnp.uint32)
    k49, _ = jax.lax.fori_loop(0, 32, _bs, (lo0, hi0))

    orig = jnp.where(
        k49 < jnp.uint32(0x80000000),
        jnp.bitwise_not(k49),
        jnp.bitwise_xor(k49, jnp.uint32(0x80000000)),
    )
    v49 = jax.lax.bitcast_convert_type(orig, jnp.float32)

    s_t = jnp.sum(jnp.where(x >= v49, e, 0.0), axis=1, keepdims=True)
    e49 = jnp.exp(v49 - l_max)
    mainpath = (s_t - e49) <= pz

    iota = jax.lax.broadcasted_iota(jnp.int32, x.shape, 1)
    x0_main = jnp.max(jnp.where(x == l_max, iota, -1), axis=1, keepdims=True)

    conf = 1.0 / s_t
    conf_ref[...] = conf
    conf2_ref[...] = conf
    idx_ref[...] = x0_main

    @pl.when(jnp.logical_not(jnp.all(mainpath)))
    def _rare():
        def _grp(i, st):
            cur_v, s_before, n_kept, d_acc, kstar, done = st
            c = jnp.sum((x == cur_v).astype(jnp.float32), axis=1, keepdims=True)
            e_v = jnp.exp(cur_v - l_max)
            q = jnp.where(
                e_v > 0.0,
                jnp.floor((pz - s_before) / e_v) + 1.0,
                jnp.where(s_before <= pz, c, 0.0),
            )
            k_g = jnp.clip(q, 0.0, c)
            k_g = jnp.where(n_kept == 0.0, jnp.maximum(k_g, 1.0), k_g)
            k_g = jnp.where(done > 0.0, 0.0, k_g)
            d_acc = d_acc + k_g * e_v
            kstar = jnp.where(i == 0, k_g, kstar)
            done = jnp.maximum(done, (k_g < c).astype(jnp.float32))
            n_kept = n_kept + k_g
            s_before = s_before + c * e_v
            nxt = jnp.max(jnp.where(x < cur_v, x, -jnp.inf), axis=1, keepdims=True)
            cur_v = jnp.where(done > 0.0, cur_v, nxt)
            return cur_v, s_before, n_kept, d_acc, kstar, done

        zero = jnp.zeros((r, 1), jnp.float32)
        st0 = (l_max, zero, zero, zero, zero, zero)
        _, _, _, d_acc, kstar, _ = jax.lax.fori_loop(0, 64, _grp, st0)

        def _ibs(_, lohi):
            lo_i, hi_i = lohi
            mid = (lo_i + hi_i) // 2
            cnt = jnp.sum(
                ((x == l_max) & (iota <= mid)).astype(jnp.float32),
                axis=1, keepdims=True)
            ok = cnt >= kstar
            return jnp.where(ok, lo_i, mid), jnp.where(ok, mid, hi_i)

        ilo0 = jnp.full((r, 1), -1, jnp.int32)
        ihi0 = jnp.full((r, 1), x.shape[1] - 1, jnp.int32)
        _, x0_rare = jax.lax.fori_loop(0, 17, _ibs, (ilo0, ihi0))

        confr = jnp.where(mainpath, conf, 1.0 / d_acc)
        conf_ref[...] = confr
        conf2_ref[...] = confr
        idx_ref[...] = jnp.where(mainpath, x0_main, x0_rare)


def _run(x_padded, n, interpret=False):
    vpad = x_padded.shape[1]
    conf, idx, conf2 = pl.pallas_call(
        _sampler_block,
        grid=(n // _ROWS_PER_BLOCK,),
        in_specs=[pl.BlockSpec((_ROWS_PER_BLOCK, vpad), lambda i: (i, 0))],
        out_specs=[
            pl.BlockSpec((_ROWS_PER_BLOCK, 1), lambda i: (i, 0)),
            pl.BlockSpec((_ROWS_PER_BLOCK, 1), lambda i: (i, 0)),
            pl.BlockSpec((_ROWS_PER_BLOCK, 1), lambda i: (i, 0)),
        ],
        out_shape=[
            jax.ShapeDtypeStruct((n, 1), jnp.float32),
            jax.ShapeDtypeStruct((n, 1), jnp.int32),
            jax.ShapeDtypeStruct((n, 1), jnp.float32),
        ],
        interpret=interpret,
    )(x_padded)
    return conf[:, 0], idx[:, 0], conf2[:, 0]


def kernel(logits):
    n, v = logits.shape
    vpad = ((v + 1023) // 1024) * 1024
    x = jnp.pad(logits, ((0, 0), (0, vpad - v)), constant_values=-jnp.inf)
    return _run(x, n)


# speedup vs baseline: 237.3422x; 1.4730x over previous
import jax
import jax.numpy as jnp
from jax.experimental import pallas as pl
from jax.experimental.pallas import tpu as pltpu

_TOP_P = 0.9
_TOP_K = 50
_ROWS_PER_BLOCK = 8
_DEPTH = 6
_SUBCH = 8


def _keyify(x):
    kb = jax.lax.bitcast_convert_type(x, jnp.uint32)
    return jnp.where(
        kb >= jnp.uint32(0x80000000),
        jnp.bitwise_not(kb),
        jnp.bitwise_or(kb, jnp.uint32(0x80000000)),
    )


def _unkey(k):
    orig = jnp.where(
        k < jnp.uint32(0x80000000),
        jnp.bitwise_not(k),
        jnp.bitwise_xor(k, jnp.uint32(0x80000000)),
    )
    return jax.lax.bitcast_convert_type(orig, jnp.float32)


def _kth_largest_key(key, k, iters):
    axes = tuple(range(1, key.ndim))

    def _bs(_, lohi):
        lo, hi = lohi
        mid = lo + ((hi - lo) >> jnp.uint32(1))
        cnt = jnp.sum((key >= mid).astype(jnp.float32), axis=axes,
                      keepdims=True)
        ge = cnt >= k
        return jnp.where(ge, mid, lo), jnp.where(ge, hi, mid)

    shp = (key.shape[0],) + (1,) * (key.ndim - 1)
    lo0 = jnp.zeros(shp, jnp.uint32)
    hi0 = jnp.full(shp, 0xFFFFFFFF, jnp.uint32)
    lo, _ = jax.lax.fori_loop(0, iters, _bs, (lo0, hi0))
    return lo.reshape((key.shape[0], 1, 1))


def _sampler_block(x_ref, conf_ref, idx_ref, conf2_ref, v49_ref, st_ref):
    x = x_ref[...]
    r, c128, _ = x.shape

    ninf = jnp.full((r, 128), -jnp.inf, jnp.float32)

    def _ins(ci, stacks):
        sl = x_ref[:, pl.ds(ci * _SUBCH, _SUBCH), :]
        for j in range(_SUBCH):
            t = sl[:, j, :]
            new = []
            for s in stacks:
                hi = jnp.maximum(s, t)
                t = jnp.minimum(s, t)
                new.append(hi)
            stacks = tuple(new)
        return stacks

    stacks = jax.lax.fori_loop(0, c128 // _SUBCH, _ins, (ninf,) * _DEPTH)
    l_max = jnp.max(stacks[0], axis=1).reshape(r, 1, 1)

    stk = jnp.concatenate([s[:, None, :] for s in stacks], axis=1)
    skey = _keyify(stk)
    cand = _unkey(_kth_largest_key(skey, _TOP_K, 32))

    e = jnp.exp(x - l_max)
    z = jnp.sum(e, axis=(1, 2), keepdims=True)
    pz = _TOP_P * z
    iota = (jax.lax.broadcasted_iota(jnp.int32, x.shape, 1) * 128
            + jax.lax.broadcasted_iota(jnp.int32, x.shape, 2))
    x0_main = jnp.max(jnp.where(x == l_max, iota, -1), axis=(1, 2),
                      keepdims=True)

    keep_c = x >= cand
    n_full = jnp.sum(keep_c.astype(jnp.float32), axis=(1, 2), keepdims=True)
    s_t_c = jnp.sum(jnp.where(keep_c, e, 0.0), axis=(1, 2), keepdims=True)
    n_stk = jnp.sum((stk >= cand).astype(jnp.float32), axis=(1, 2),
                    keepdims=True)
    verified = n_stk == n_full

    v49_ref[...] = cand[:, :, 0]
    st_ref[...] = s_t_c[:, :, 0]

    @pl.when(jnp.logical_not(jnp.all(verified)))
    def _unverified():
        v49_f = _unkey(_kth_largest_key(_keyify(x), _TOP_K, 32))
        s_t_f = jnp.sum(jnp.where(x >= v49_f, e, 0.0), axis=(1, 2),
                        keepdims=True)
        v49_ref[...] = jnp.where(verified, cand, v49_f)[:, :, 0]
        st_ref[...] = jnp.where(verified, s_t_c, s_t_f)[:, :, 0]

    v49 = v49_ref[...][:, :, None]
    s_t = st_ref[...][:, :, None]

    e49 = jnp.exp(v49 - l_max)
    mainpath = (s_t - e49) <= pz

    conf = 1.0 / s_t
    conf_ref[...] = conf[:, :, 0]
    conf2_ref[...] = conf[:, :, 0]
    idx_ref[...] = x0_main[:, :, 0]

    @pl.when(jnp.logical_not(jnp.all(mainpath)))
    def _rare():
        def _grp(i, st):
            cur_v, s_before, n_kept, d_acc, kstar, done = st
            c = jnp.sum((x == cur_v).astype(jnp.float32), axis=(1, 2),
                        keepdims=True)
            e_v = jnp.exp(cur_v - l_max)
            q = jnp.where(
                e_v > 0.0,
                jnp.floor((pz - s_before) / e_v) + 1.0,
                jnp.where(s_before <= pz, c, 0.0),
            )
            k_g = jnp.clip(q, 0.0, c)
            k_g = jnp.where(n_kept == 0.0, jnp.maximum(k_g, 1.0), k_g)
            k_g = jnp.where(done > 0.0, 0.0, k_g)
            d_acc = d_acc + k_g * e_v
            kstar = jnp.where(i == 0, k_g, kstar)
            done = jnp.maximum(done, (k_g < c).astype(jnp.float32))
            n_kept = n_kept + k_g
            s_before = s_before + c * e_v
            nxt = jnp.max(jnp.where(x < cur_v, x, -jnp.inf), axis=(1, 2),
                          keepdims=True)
            cur_v = jnp.where(done > 0.0, cur_v, nxt)
            return cur_v, s_before, n_kept, d_acc, kstar, done

        zero = jnp.zeros((r, 1, 1), jnp.float32)
        st0 = (l_max, zero, zero, zero, zero, zero)
        _, _, _, d_acc, kstar, _ = jax.lax.fori_loop(0, 64, _grp, st0)

        def _ibs(_, lohi):
            lo_i, hi_i = lohi
            mid = (lo_i + hi_i) // 2
            cnt = jnp.sum(
                ((x == l_max) & (iota <= mid)).astype(jnp.float32),
                axis=(1, 2), keepdims=True)
            ok = cnt >= kstar
            return jnp.where(ok, lo_i, mid), jnp.where(ok, mid, hi_i)

        ilo0 = jnp.full((r, 1, 1), -1, jnp.int32)
        ihi0 = jnp.full((r, 1, 1), c128 * 128 - 1, jnp.int32)
        _, x0_rare = jax.lax.fori_loop(0, 17, _ibs, (ilo0, ihi0))

        confr = jnp.where(mainpath, conf, 1.0 / d_acc)
        conf_ref[...] = confr[:, :, 0]
        conf2_ref[...] = confr[:, :, 0]
        idx_ref[...] = jnp.where(mainpath, x0_main, x0_rare)[:, :, 0]


def _run(x3, n, interpret=False):
    c128 = x3.shape[1]
    conf, idx, conf2 = pl.pallas_call(
        _sampler_block,
        grid=(n // _ROWS_PER_BLOCK,),
        in_specs=[pl.BlockSpec((_ROWS_PER_BLOCK, c128, 128),
                               lambda i: (i, 0, 0))],
        out_specs=[
            pl.BlockSpec((_ROWS_PER_BLOCK, 1), lambda i: (i, 0)),
            pl.BlockSpec((_ROWS_PER_BLOCK, 1), lambda i: (i, 0)),
            pl.BlockSpec((_ROWS_PER_BLOCK, 1), lambda i: (i, 0)),
        ],
        out_shape=[
            jax.ShapeDtypeStruct((n, 1), jnp.float32),
            jax.ShapeDtypeStruct((n, 1), jnp.int32),
            jax.ShapeDtypeStruct((n, 1), jnp.float32),
        ],
        scratch_shapes=[
            pltpu.VMEM((_ROWS_PER_BLOCK, 1), jnp.float32),
            pltpu.VMEM((_ROWS_PER_BLOCK, 1), jnp.float32),
        ],
        compiler_params=pltpu.CompilerParams(
            dimension_semantics=("parallel",)),
        interpret=interpret,
    )(x3)
    return conf[:, 0], idx[:, 0], conf2[:, 0]


def kernel(logits):
    n, v = logits.shape
    vpad = ((v + 1023) // 1024) * 1024
    x = jnp.pad(logits, ((0, 0), (0, vpad - v)), constant_values=-jnp.inf)
    return _run(x.reshape(n, vpad // 128, 128), n)


# speedup vs baseline: 525.7817x; 2.2153x over previous
import jax
import jax.numpy as jnp
from jax.experimental import pallas as pl
from jax.experimental.pallas import tpu as pltpu

_TOP_P = 0.9
_TOP_K = 50
_ROWS_PER_BLOCK = 8
_DEPTH = 5
_NSETS = 4


def _keyify(x):
    kb = jax.lax.bitcast_convert_type(x, jnp.uint32)
    return jnp.where(
        kb >= jnp.uint32(0x80000000),
        jnp.bitwise_not(kb),
        jnp.bitwise_or(kb, jnp.uint32(0x80000000)),
    )


def _unkey(k):
    orig = jnp.where(
        k < jnp.uint32(0x80000000),
        jnp.bitwise_not(k),
        jnp.bitwise_xor(k, jnp.uint32(0x80000000)),
    )
    return jax.lax.bitcast_convert_type(orig, jnp.float32)


def _kth_largest_key(key, k, iters):
    def _bs(_, lohi):
        lo, hi = lohi
        mid = lo + ((hi - lo) >> jnp.uint32(1))
        cnt = jnp.sum((key >= mid).astype(jnp.float32), axis=1,
                      keepdims=True)
        ge = cnt >= k
        return jnp.where(ge, mid, lo), jnp.where(ge, hi, mid)

    shp = (key.shape[0], 1)
    lo0 = jnp.zeros(shp, jnp.uint32)
    hi0 = jnp.full(shp, 0xFFFFFFFF, jnp.uint32)
    lo, _ = jax.lax.fori_loop(0, iters, _bs, (lo0, hi0))
    return lo


def _sampler_block(x_ref, conf_ref, idx_ref, conf2_ref, v49_ref, st_ref):
    x = x_ref[...]
    r, vpad = x.shape
    nch = vpad // 128

    ninf = jnp.full((r, 128), -jnp.inf, jnp.float32)

    def _ins(ci, sets):
        sl = x_ref[:, pl.ds(ci * (128 * _NSETS), 128 * _NSETS)]
        new_sets = []
        for j, stack in enumerate(sets):
            t = sl[:, j * 128:(j + 1) * 128]
            new = []
            for s in stack:
                hi = jnp.maximum(s, t)
                t = jnp.minimum(s, t)
                new.append(hi)
            new_sets.append(tuple(new))
        return tuple(new_sets)

    sets0 = tuple((ninf,) * _DEPTH for _ in range(_NSETS))
    sets = jax.lax.fori_loop(0, nch // _NSETS, _ins, sets0)
    flat = [s for stack in sets for s in stack]
    stk = jnp.concatenate(flat, axis=1)
    l_max = jnp.max(stk, axis=1, keepdims=True)

    cand = _unkey(_kth_largest_key(_keyify(stk), _TOP_K, 32))

    e = jnp.exp(x - l_max)
    z = jnp.sum(e, axis=1, keepdims=True)
    pz = _TOP_P * z
    iota = jax.lax.broadcasted_iota(jnp.int32, x.shape, 1)
    x0_main = jnp.max(jnp.where(x == l_max, iota, -1), axis=1, keepdims=True)

    keep_c = x >= cand
    n_full = jnp.sum(keep_c.astype(jnp.float32), axis=1, keepdims=True)
    s_t_c = jnp.sum(jnp.where(keep_c, e, 0.0), axis=1, keepdims=True)
    n_stk = jnp.sum((stk >= cand).astype(jnp.float32), axis=1, keepdims=True)
    verified = n_stk == n_full

    v49_ref[...] = cand
    st_ref[...] = s_t_c

    @pl.when(jnp.logical_not(jnp.all(verified)))
    def _unverified():
        v49_f = _unkey(_kth_largest_key(_keyify(x), _TOP_K, 32))
        s_t_f = jnp.sum(jnp.where(x >= v49_f, e, 0.0), axis=1, keepdims=True)
        v49_ref[...] = jnp.where(verified, cand, v49_f)
        st_ref[...] = jnp.where(verified, s_t_c, s_t_f)

    v49 = v49_ref[...]
    s_t = st_ref[...]

    e49 = jnp.exp(v49 - l_max)
    mainpath = (s_t - e49) <= pz

    conf = 1.0 / s_t
    conf_ref[...] = conf
    conf2_ref[...] = conf
    idx_ref[...] = x0_main

    @pl.when(jnp.logical_not(jnp.all(mainpath)))
    def _rare():
        def _grp(i, st):
            cur_v, s_before, n_kept, d_acc, kstar, done = st
            c = jnp.sum((x == cur_v).astype(jnp.float32), axis=1,
                        keepdims=True)
            e_v = jnp.exp(cur_v - l_max)
            q = jnp.where(
                e_v > 0.0,
                jnp.floor((pz - s_before) / e_v) + 1.0,
                jnp.where(s_before <= pz, c, 0.0),
            )
            k_g = jnp.clip(q, 0.0, c)
            k_g = jnp.where(n_kept == 0.0, jnp.maximum(k_g, 1.0), k_g)
            k_g = jnp.where(done > 0.0, 0.0, k_g)
            d_acc = d_acc + k_g * e_v
            kstar = jnp.where(i == 0, k_g, kstar)
            done = jnp.maximum(done, (k_g < c).astype(jnp.float32))
            n_kept = n_kept + k_g
            s_before = s_before + c * e_v
            nxt = jnp.max(jnp.where(x < cur_v, x, -jnp.inf), axis=1,
                          keepdims=True)
            cur_v = jnp.where(done > 0.0, cur_v, nxt)
            return cur_v, s_before, n_kept, d_acc, kstar, done

        zero = jnp.zeros((r, 1), jnp.float32)
        st0 = (l_max, zero, zero, zero, zero, zero)
        _, _, _, d_acc, kstar, _ = jax.lax.fori_loop(0, 64, _grp, st0)

        def _ibs(_, lohi):
            lo_i, hi_i = lohi
            mid = (lo_i + hi_i) // 2
            cnt = jnp.sum(
                ((x == l_max) & (iota <= mid)).astype(jnp.float32),
                axis=1, keepdims=True)
            ok = cnt >= kstar
            return jnp.where(ok, lo_i, mid), jnp.where(ok, mid, hi_i)

        ilo0 = jnp.full((r, 1), -1, jnp.int32)
        ihi0 = jnp.full((r, 1), vpad - 1, jnp.int32)
        _, x0_rare = jax.lax.fori_loop(0, 17, _ibs, (ilo0, ihi0))

        confr = jnp.where(mainpath, conf, 1.0 / d_acc)
        conf_ref[...] = confr
        conf2_ref[...] = confr
        idx_ref[...] = jnp.where(mainpath, x0_main, x0_rare)


def _run(x_padded, n, interpret=False):
    vpad = x_padded.shape[1]
    conf, idx, conf2 = pl.pallas_call(
        _sampler_block,
        grid=(n // _ROWS_PER_BLOCK,),
        in_specs=[pl.BlockSpec((_ROWS_PER_BLOCK, vpad), lambda i: (i, 0))],
        out_specs=[
            pl.BlockSpec((_ROWS_PER_BLOCK, 1), lambda i: (i, 0)),
            pl.BlockSpec((_ROWS_PER_BLOCK, 1), lambda i: (i, 0)),
            pl.BlockSpec((_ROWS_PER_BLOCK, 1), lambda i: (i, 0)),
        ],
        out_shape=[
            jax.ShapeDtypeStruct((n, 1), jnp.float32),
            jax.ShapeDtypeStruct((n, 1), jnp.int32),
            jax.ShapeDtypeStruct((n, 1), jnp.float32),
        ],
        scratch_shapes=[
            pltpu.VMEM((_ROWS_PER_BLOCK, 1), jnp.float32),
            pltpu.VMEM((_ROWS_PER_BLOCK, 1), jnp.float32),
        ],
        compiler_params=pltpu.CompilerParams(
            dimension_semantics=("parallel",)),
        interpret=interpret,
    )(x_padded)
    return conf[:, 0], idx[:, 0], conf2[:, 0]


def kernel(logits):
    n, v = logits.shape
    vpad = ((v + 511) // 512) * 512
    x = jnp.pad(logits, ((0, 0), (0, vpad - v)), constant_values=-jnp.inf)
    return _run(x, n)


# speedup vs baseline: 665.2056x; 1.2652x over previous
import jax
import jax.numpy as jnp
from jax.experimental import pallas as pl
from jax.experimental.pallas import tpu as pltpu

_TOP_P = 0.9
_TOP_K = 50
_ROWS_PER_BLOCK = 8
_DEPTH = 5
_NSETS = 4


def _keyify(x):
    kb = jax.lax.bitcast_convert_type(x, jnp.uint32)
    return jnp.where(
        kb >= jnp.uint32(0x80000000),
        jnp.bitwise_not(kb),
        jnp.bitwise_or(kb, jnp.uint32(0x80000000)),
    )


def _unkey(k):
    orig = jnp.where(
        k < jnp.uint32(0x80000000),
        jnp.bitwise_not(k),
        jnp.bitwise_xor(k, jnp.uint32(0x80000000)),
    )
    return jax.lax.bitcast_convert_type(orig, jnp.float32)


def _kth_largest_key(key, k, iters):
    def _bs(_, lohi):
        lo, hi = lohi
        mid = lo + ((hi - lo) >> jnp.uint32(1))
        cnt = jnp.sum((key >= mid).astype(jnp.float32), axis=1,
                      keepdims=True)
        ge = cnt >= k
        return jnp.where(ge, mid, lo), jnp.where(ge, hi, mid)

    shp = (key.shape[0], 1)
    lo0 = jnp.zeros(shp, jnp.uint32)
    hi0 = jnp.full(shp, 0xFFFFFFFF, jnp.uint32)
    lo, _ = jax.lax.fori_loop(0, iters, _bs, (lo0, hi0))
    return lo


def _sampler_block(x_ref, conf_ref, idx_ref, conf2_ref, v49_ref, st_ref):
    x = x_ref[...]
    r, v = x.shape

    ninf = jnp.full((r, 128), -jnp.inf, jnp.float32)

    def _insert(stack, t):
        new = []
        for s in stack:
            hi = jnp.maximum(s, t)
            t = jnp.minimum(s, t)
            new.append(hi)
        return tuple(new)

    def _ins(ci, sets):
        sl = x_ref[:, pl.ds(ci * (128 * _NSETS), 128 * _NSETS)]
        return tuple(
            _insert(stack, sl[:, j * 128:(j + 1) * 128])
            for j, stack in enumerate(sets)
        )

    sets0 = tuple((ninf,) * _DEPTH for _ in range(_NSETS))
    sets = jax.lax.fori_loop(0, v // (128 * _NSETS), _ins, sets0)

    sets = list(sets)
    for j, off in enumerate(range(v - v % (128 * _NSETS), v, 128)):
        w = min(128, v - off)
        t = x_ref[:, off:off + w]
        if w < 128:
            t = jnp.concatenate(
                [t, jnp.full((r, 128 - w), -jnp.inf, jnp.float32)], axis=1)
        sets[j % _NSETS] = _insert(sets[j % _NSETS], t)

    flat = [s for stack in sets for s in stack]
    stk = jnp.concatenate(flat, axis=1)
    l_max = jnp.max(stk, axis=1, keepdims=True)

    cand = _unkey(_kth_largest_key(_keyify(stk), _TOP_K, 32))

    e = jnp.exp(x - l_max)
    z = jnp.sum(e, axis=1, keepdims=True)
    pz = _TOP_P * z
    iota = jax.lax.broadcasted_iota(jnp.int32, x.shape, 1)
    x0_main = jnp.max(jnp.where(x == l_max, iota, -1), axis=1, keepdims=True)

    keep_c = x >= cand
    n_full = jnp.sum(keep_c.astype(jnp.float32), axis=1, keepdims=True)
    s_t_c = jnp.sum(jnp.where(keep_c, e, 0.0), axis=1, keepdims=True)
    n_stk = jnp.sum((stk >= cand).astype(jnp.float32), axis=1, keepdims=True)
    verified = n_stk == n_full

    v49_ref[...] = cand
    st_ref[...] = s_t_c

    @pl.when(jnp.logical_not(jnp.all(verified)))
    def _unverified():
        v49_f = _unkey(_kth_largest_key(_keyify(x), _TOP_K, 32))
        s_t_f = jnp.sum(jnp.where(x >= v49_f, e, 0.0), axis=1, keepdims=True)
        v49_ref[...] = jnp.where(verified, cand, v49_f)
        st_ref[...] = jnp.where(verified, s_t_c, s_t_f)

    v49 = v49_ref[...]
    s_t = st_ref[...]

    e49 = jnp.exp(v49 - l_max)
    mainpath = (s_t - e49) <= pz

    conf = 1.0 / s_t
    conf_ref[...] = conf
    conf2_ref[...] = conf
    idx_ref[...] = x0_main

    @pl.when(jnp.logical_not(jnp.all(mainpath)))
    def _rare():
        def _grp(i, st):
            cur_v, s_before, n_kept, d_acc, kstar, done = st
            c = jnp.sum((x == cur_v).astype(jnp.float32), axis=1,
                        keepdims=True)
            e_v = jnp.exp(cur_v - l_max)
            q = jnp.where(
                e_v > 0.0,
                jnp.floor((pz - s_before) / e_v) + 1.0,
                jnp.where(s_before <= pz, c, 0.0),
            )
            k_g = jnp.clip(q, 0.0, c)
            k_g = jnp.where(n_kept == 0.0, jnp.maximum(k_g, 1.0), k_g)
            k_g = jnp.where(done > 0.0, 0.0, k_g)
            d_acc = d_acc + k_g * e_v
            kstar = jnp.where(i == 0, k_g, kstar)
            done = jnp.maximum(done, (k_g < c).astype(jnp.float32))
            n_kept = n_kept + k_g
            s_before = s_before + c * e_v
            nxt = jnp.max(jnp.where(x < cur_v, x, -jnp.inf), axis=1,
                          keepdims=True)
            cur_v = jnp.where(done > 0.0, cur_v, nxt)
            return cur_v, s_before, n_kept, d_acc, kstar, done

        zero = jnp.zeros((r, 1), jnp.float32)
        st0 = (l_max, zero, zero, zero, zero, zero)
        _, _, _, d_acc, kstar, _ = jax.lax.fori_loop(0, 64, _grp, st0)

        def _ibs(_, lohi):
            lo_i, hi_i = lohi
            mid = (lo_i + hi_i) // 2
            cnt = jnp.sum(
                ((x == l_max) & (iota <= mid)).astype(jnp.float32),
                axis=1, keepdims=True)
            ok = cnt >= kstar
            return jnp.where(ok, lo_i, mid), jnp.where(ok, mid, hi_i)

        ilo0 = jnp.full((r, 1), -1, jnp.int32)
        ihi0 = jnp.full((r, 1), v - 1, jnp.int32)
        _, x0_rare = jax.lax.fori_loop(0, 17, _ibs, (ilo0, ihi0))

        confr = jnp.where(mainpath, conf, 1.0 / d_acc)
        conf_ref[...] = confr
        conf2_ref[...] = confr
        idx_ref[...] = jnp.where(mainpath, x0_main, x0_rare)


def _run(x, n, interpret=False):
    v = x.shape[1]
    conf, idx, conf2 = pl.pallas_call(
        _sampler_block,
        grid=(n // _ROWS_PER_BLOCK,),
        in_specs=[pl.BlockSpec((_ROWS_PER_BLOCK, v), lambda i: (i, 0))],
        out_specs=[
            pl.BlockSpec((_ROWS_PER_BLOCK, 1), lambda i: (i, 0)),
            pl.BlockSpec((_ROWS_PER_BLOCK, 1), lambda i: (i, 0)),
            pl.BlockSpec((_ROWS_PER_BLOCK, 1), lambda i: (i, 0)),
        ],
        out_shape=[
            jax.ShapeDtypeStruct((n, 1), jnp.float32),
            jax.ShapeDtypeStruct((n, 1), jnp.int32),
            jax.ShapeDtypeStruct((n, 1), jnp.float32),
        ],
        scratch_shapes=[
            pltpu.VMEM((_ROWS_PER_BLOCK, 1), jnp.float32),
            pltpu.VMEM((_ROWS_PER_BLOCK, 1), jnp.float32),
        ],
        compiler_params=pltpu.CompilerParams(
            dimension_semantics=("parallel",)),
        interpret=interpret,
    )(x)
    return conf[:, 0], idx[:, 0], conf2[:, 0]


def kernel(logits):
    return _run(logits, logits.shape[0])


# speedup vs baseline: 665.2444x; 1.0001x over previous
import jax
import jax.numpy as jnp
from jax.experimental import pallas as pl
from jax.experimental.pallas import tpu as pltpu

_TOP_P = 0.9
_TOP_K = 50
_ROWS_PER_BLOCK = 16
_DEPTH = 4
_NSETS = 8


def _keyify(x):
    kb = jax.lax.bitcast_convert_type(x, jnp.uint32)
    return jnp.where(
        kb >= jnp.uint32(0x80000000),
        jnp.bitwise_not(kb),
        jnp.bitwise_or(kb, jnp.uint32(0x80000000)),
    )


def _unkey(k):
    orig = jnp.where(
        k < jnp.uint32(0x80000000),
        jnp.bitwise_not(k),
        jnp.bitwise_xor(k, jnp.uint32(0x80000000)),
    )
    return jax.lax.bitcast_convert_type(orig, jnp.float32)


def _kth_largest_key(key, k, iters):
    def _bs(_, lohi):
        lo, hi = lohi
        mid = lo + ((hi - lo) >> jnp.uint32(1))
        cnt = jnp.sum((key >= mid).astype(jnp.float32), axis=1,
                      keepdims=True)
        ge = cnt >= k
        return jnp.where(ge, mid, lo), jnp.where(ge, hi, mid)

    shp = (key.shape[0], 1)
    lo0 = jnp.zeros(shp, jnp.uint32)
    hi0 = jnp.full(shp, 0xFFFFFFFF, jnp.uint32)
    lo, _ = jax.lax.fori_loop(0, iters, _bs, (lo0, hi0))
    return lo


def _sampler_block(x_ref, conf_ref, idx_ref, conf2_ref, v49_ref, st_ref):
    x = x_ref[...]
    r, v = x.shape

    ninf = jnp.full((r, 128), -jnp.inf, jnp.float32)

    def _insert(stack, t):
        new = []
        for s in stack:
            hi = jnp.maximum(s, t)
            t = jnp.minimum(s, t)
            new.append(hi)
        return tuple(new)

    def _ins(ci, sets):
        sl = x_ref[:, pl.ds(ci * (128 * _NSETS), 128 * _NSETS)]
        return tuple(
            _insert(stack, sl[:, j * 128:(j + 1) * 128])
            for j, stack in enumerate(sets)
        )

    sets0 = tuple((ninf,) * _DEPTH for _ in range(_NSETS))
    sets = jax.lax.fori_loop(0, v // (128 * _NSETS), _ins, sets0)

    sets = list(sets)
    for j, off in enumerate(range(v - v % (128 * _NSETS), v, 128)):
        w = min(128, v - off)
        t = x_ref[:, off:off + w]
        if w < 128:
            t = jnp.concatenate(
                [t, jnp.full((r, 128 - w), -jnp.inf, jnp.float32)], axis=1)
        sets[j % _NSETS] = _insert(sets[j % _NSETS], t)

    flat = [s for stack in sets for s in stack]
    stk = jnp.concatenate(flat, axis=1)
    l_max = jnp.max(stk, axis=1, keepdims=True)

    cand = _unkey(_kth_largest_key(_keyify(stk), _TOP_K, 32))

    e = jnp.exp(x - l_max)
    z = jnp.sum(e, axis=1, keepdims=True)
    pz = _TOP_P * z
    iota = jax.lax.broadcasted_iota(jnp.int32, x.shape, 1)
    x0_main = jnp.max(jnp.where(x == l_max, iota, -1), axis=1, keepdims=True)

    keep_c = x >= cand
    n_full = jnp.sum(keep_c.astype(jnp.float32), axis=1, keepdims=True)
    s_t_c = jnp.sum(jnp.where(keep_c, e, 0.0), axis=1, keepdims=True)
    n_stk = jnp.sum((stk >= cand).astype(jnp.float32), axis=1, keepdims=True)
    verified = n_stk == n_full

    v49_ref[...] = cand
    st_ref[...] = s_t_c

    @pl.when(jnp.logical_not(jnp.all(verified)))
    def _unverified():
        v49_f = _unkey(_kth_largest_key(_keyify(x), _TOP_K, 32))
        s_t_f = jnp.sum(jnp.where(x >= v49_f, e, 0.0), axis=1, keepdims=True)
        v49_ref[...] = jnp.where(verified, cand, v49_f)
        st_ref[...] = jnp.where(verified, s_t_c, s_t_f)

    v49 = v49_ref[...]
    s_t = st_ref[...]

    e49 = jnp.exp(v49 - l_max)
    mainpath = (s_t - e49) <= pz

    conf = 1.0 / s_t
    conf_ref[...] = conf
    conf2_ref[...] = conf
    idx_ref[...] = x0_main

    @pl.when(jnp.logical_not(jnp.all(mainpath)))
    def _rare():
        def _grp(i, st):
            cur_v, s_before, n_kept, d_acc, kstar, done = st
            c = jnp.sum((x == cur_v).astype(jnp.float32), axis=1,
                        keepdims=True)
            e_v = jnp.exp(cur_v - l_max)
            q = jnp.where(
                e_v > 0.0,
                jnp.floor((pz - s_before) / e_v) + 1.0,
                jnp.where(s_before <= pz, c, 0.0),
            )
            k_g = jnp.clip(q, 0.0, c)
            k_g = jnp.where(n_kept == 0.0, jnp.maximum(k_g, 1.0), k_g)
            k_g = jnp.where(done > 0.0, 0.0, k_g)
            d_acc = d_acc + k_g * e_v
            kstar = jnp.where(i == 0, k_g, kstar)
            done = jnp.maximum(done, (k_g < c).astype(jnp.float32))
            n_kept = n_kept + k_g
            s_before = s_before + c * e_v
            nxt = jnp.max(jnp.where(x < cur_v, x, -jnp.inf), axis=1,
                          keepdims=True)
            cur_v = jnp.where(done > 0.0, cur_v, nxt)
            return cur_v, s_before, n_kept, d_acc, kstar, done

        zero = jnp.zeros((r, 1), jnp.float32)
        st0 = (l_max, zero, zero, zero, zero, zero)
        _, _, _, d_acc, kstar, _ = jax.lax.fori_loop(0, 64, _grp, st0)

        def _ibs(_, lohi):
            lo_i, hi_i = lohi
            mid = (lo_i + hi_i) // 2
            cnt = jnp.sum(
                ((x == l_max) & (iota <= mid)).astype(jnp.float32),
                axis=1, keepdims=True)
            ok = cnt >= kstar
            return jnp.where(ok, lo_i, mid), jnp.where(ok, mid, hi_i)

        ilo0 = jnp.full((r, 1), -1, jnp.int32)
        ihi0 = jnp.full((r, 1), v - 1, jnp.int32)
        _, x0_rare = jax.lax.fori_loop(0, 17, _ibs, (ilo0, ihi0))

        confr = jnp.where(mainpath, conf, 1.0 / d_acc)
        conf_ref[...] = confr
        conf2_ref[...] = confr
        idx_ref[...] = jnp.where(mainpath, x0_main, x0_rare)


def _run(x, n, interpret=False):
    v = x.shape[1]
    conf, idx, conf2 = pl.pallas_call(
        _sampler_block,
        grid=(n // _ROWS_PER_BLOCK,),
        in_specs=[pl.BlockSpec((_ROWS_PER_BLOCK, v), lambda i: (i, 0))],
        out_specs=[
            pl.BlockSpec((_ROWS_PER_BLOCK, 1), lambda i: (i, 0)),
            pl.BlockSpec((_ROWS_PER_BLOCK, 1), lambda i: (i, 0)),
            pl.BlockSpec((_ROWS_PER_BLOCK, 1), lambda i: (i, 0)),
        ],
        out_shape=[
            jax.ShapeDtypeStruct((n, 1), jnp.float32),
            jax.ShapeDtypeStruct((n, 1), jnp.int32),
            jax.ShapeDtypeStruct((n, 1), jnp.float32),
        ],
        scratch_shapes=[
            pltpu.VMEM((_ROWS_PER_BLOCK, 1), jnp.float32),
            pltpu.VMEM((_ROWS_PER_BLOCK, 1), jnp.float32),
        ],
        compiler_params=pltpu.CompilerParams(
            dimension_semantics=("parallel",)),
        interpret=interpret,
    )(x)
    return conf[:, 0], idx[:, 0], conf2[:, 0]


def kernel(logits):
    return _run(logits, logits.shape[0])
